# baseline (speedup 1.0000x reference)
import functools

import jax
import jax.numpy as jnp
from jax.experimental import pallas as pl


def _tree_body(depth, B, D, H, R,
               nf_ref, rel_ref, rel_emb_ref, watt_ref, Wfcat_ref, bf_ref,
               Wxcat_ref, Whcat_ref, bcat_ref,
               out_ref):
    f32 = jnp.float32

    def mm(a, b):
        return jnp.dot(a, b, preferred_element_type=f32)

    def sig(x):
        return 0.5 * jnp.tanh(0.5 * x) + 0.5

    def gates(zcat, c_sum):
        i_g = sig(zcat[:, :H])
        o_g = sig(zcat[:, H:2 * H])
        u_g = jnp.tanh(zcat[:, 2 * H:])
        c = i_g * u_g + c_sum
        h = o_g * jnp.tanh(c)
        return h, c

    n = 2 ** (depth - 1)
    xf = nf_ref[n - 1:2 * n - 1].reshape(n * B, D)
    h, c = gates(mm(xf, Wxcat_ref[...]) + bcat_ref[...], 0.0)

    watt = watt_ref[...].reshape(1, 1, H)

    for l in range(depth - 2, -1, -1):
        n = 2 ** l
        m = 2 * n
        c0 = 2 * n - 1

        h3 = h.reshape(m, B, H)
        c3 = c.reshape(m, B, H)

        rel = rel_ref[c0:c0 + m]

        def sel(rows_ref):
            out = rows_ref[R - 1:R, :].reshape(1, 1, H)
            out = jnp.broadcast_to(out, (m, 1, H))
            for r in range(R - 2, -1, -1):
                out = jnp.where(rel == r,
                                rows_ref[r:r + 1, :].reshape(1, 1, H), out)
            return out

        remb = sel(rel_emb_ref)
        bfs = sel(bf_ref)

        s = (jnp.sum(h3 * watt, axis=-1, keepdims=True)
             + jnp.sum(remb * watt, axis=-1, keepdims=True))
        s4 = s.reshape(n, 2, B, 1)
        a = sig(s4[:, 0] - s4[:, 1])
        h4 = h3.reshape(n, 2, B, H)
        h_sum = h4[:, 1] + a * (h4[:, 0] - h4[:, 1])

        fcat = mm(h, Wfcat_ref[...])
        fs = [fcat[:, r * H:(r + 1) * H].reshape(m, B, H) for r in range(R)]
        f = fs[R - 1]
        for r in range(R - 2, -1, -1):
            f = jnp.where(rel == r, fs[r], f)
        f = f + bfs
        fc = (f * c3).reshape(n, 2, B, H)
        c_sum = (fc[:, 0] + fc[:, 1]).reshape(n * B, H)

        xf = nf_ref[n - 1:2 * n - 1].reshape(n * B, D)
        hs = h_sum.reshape(n * B, H)
        h, c = gates(mm(xf, Wxcat_ref[...]) + mm(hs, Whcat_ref[...])
                     + bcat_ref[...], c_sum)

    out_ref[...] = h


def kernel(node_features, rel_emb, W_att, b_att, W_i, b_i, W_o, b_o,
           W_u, b_u, Wf, bf, W_enc, b_enc, relation_ids):
    B, N, D = node_features.shape
    R, H = rel_emb.shape
    depth = (N + 1).bit_length() - 1

    nf = jnp.transpose(node_features, (1, 0, 2)).astype(jnp.bfloat16)
    relH = jnp.broadcast_to(
        relation_ids.astype(jnp.int32)[:, None, None], (N, 1, H))
    watt = W_att.reshape(1, H)
    Wxcat = jnp.concatenate(
        [W_i[:D], W_o[:D], W_u[:D]], axis=1).astype(jnp.bfloat16)
    Whcat = jnp.concatenate([W_i[D:], W_o[D:], W_u[D:]], axis=1)
    bcat = jnp.concatenate([b_i, b_o, b_u]).reshape(1, 3 * H)
    Wfcat = jnp.transpose(Wf, (1, 0, 2)).reshape(H, R * H)

    body = functools.partial(_tree_body, depth, B, D, H, R)
    return pl.pallas_call(
        body,
        out_shape=jax.ShapeDtypeStruct((B, H), jnp.float32),
    )(nf, relH, rel_emb, watt, Wfcat, bf, Wxcat, Whcat, bcat)


# speedup vs baseline: 1.2684x; 1.2684x over previous
import functools

import jax
import jax.numpy as jnp
from jax.experimental import pallas as pl


def _tree_body(depth, B, D, H, R,
               nf_ref, rel_ref, rel_emb_ref, watt_ref, Wfcat_ref, bf_ref,
               Wxcat_ref, Whcat_ref, bcat_ref,
               out_ref):
    f32 = jnp.float32

    def mm(a, b):
        return jnp.dot(a, b, preferred_element_type=f32)

    def sig(x):
        return 0.5 * jnp.tanh(0.5 * x) + 0.5

    def gates(zcat, c_sum):
        i_g = sig(zcat[:, :H])
        o_g = sig(zcat[:, H:2 * H])
        u_g = jnp.tanh(zcat[:, 2 * H:])
        c = i_g * u_g + c_sum
        h = o_g * jnp.tanh(c)
        return h, c

    n = 2 ** (depth - 1)
    xf = nf_ref[n - 1:2 * n - 1].reshape(n * B, D)
    h, c = gates(mm(xf, Wxcat_ref[...]) + bcat_ref[...], 0.0)

    watt = watt_ref[...].reshape(1, 1, H)

    for l in range(depth - 2, -1, -1):
        n = 2 ** l
        m = 2 * n
        c0 = 2 * n - 1

        h3 = h.reshape(m, B, H)
        c3 = c.reshape(m, B, H)

        rel = rel_ref[c0:c0 + m]

        def sel(rows_ref):
            out = rows_ref[R - 1:R, :].reshape(1, 1, H)
            out = jnp.broadcast_to(out, (m, 1, H))
            for r in range(R - 2, -1, -1):
                out = jnp.where(rel == r,
                                rows_ref[r:r + 1, :].reshape(1, 1, H), out)
            return out

        remb = sel(rel_emb_ref)
        bfs = sel(bf_ref)

        h4 = h3.reshape(n, 2, B, H)
        hd = h4[:, 0] - h4[:, 1]
        remb4 = remb.reshape(n, 2, 1, H)
        rd = remb4[:, 0] - remb4[:, 1]
        d = (jnp.sum(hd * watt, axis=-1, keepdims=True)
             + jnp.sum(rd * watt, axis=-1, keepdims=True))
        a = sig(d)
        h_sum = h4[:, 1] + a * hd

        fcat = mm(h, Wfcat_ref[...])
        fs = [fcat[:, r * H:(r + 1) * H].reshape(m, B, H) for r in range(R)]
        f = fs[R - 1]
        for r in range(R - 2, -1, -1):
            f = jnp.where(rel == r, fs[r], f)
        f = f + bfs
        fc = (f * c3).reshape(n, 2, B, H)
        c_sum = (fc[:, 0] + fc[:, 1]).reshape(n * B, H)

        xf = nf_ref[n - 1:2 * n - 1].reshape(n * B, D)
        hs = h_sum.reshape(n * B, H)
        h, c = gates(mm(xf, Wxcat_ref[...]) + mm(hs, Whcat_ref[...])
                     + bcat_ref[...], c_sum)

    out_ref[...] = h


def kernel(node_features, rel_emb, W_att, b_att, W_i, b_i, W_o, b_o,
           W_u, b_u, Wf, bf, W_enc, b_enc, relation_ids):
    B, N, D = node_features.shape
    R, H = rel_emb.shape
    depth = (N + 1).bit_length() - 1

    nf = jnp.transpose(node_features, (1, 0, 2))
    relH = jnp.broadcast_to(
        relation_ids.astype(jnp.int32)[:, None, None], (N, 1, H))
    watt = W_att.reshape(1, H)
    Wxcat = jnp.concatenate([W_i[:D], W_o[:D], W_u[:D]], axis=1)
    Whcat = jnp.concatenate([W_i[D:], W_o[D:], W_u[D:]], axis=1)
    bcat = jnp.concatenate([b_i, b_o, b_u]).reshape(1, 3 * H)
    Wfcat = jnp.transpose(Wf, (1, 0, 2)).reshape(H, R * H)

    body = functools.partial(_tree_body, depth, B, D, H, R)
    return pl.pallas_call(
        body,
        out_shape=jax.ShapeDtypeStruct((B, H), jnp.float32),
    )(nf, relH, rel_emb, watt, Wfcat, bf, Wxcat, Whcat, bcat)
